# Initial kernel scaffold; baseline (speedup 1.0000x reference)
import jax
import jax.numpy as jnp
from jax.experimental import pallas as pl


def kernel(time_step, batch, edge_index):
    raise NotImplementedError("write your pallas kernel here")


# speedup vs baseline: 12.2473x; 12.2473x over previous
import functools

import jax
import jax.numpy as jnp
from jax import lax
from jax.experimental import pallas as pl
from jax.experimental.pallas import tpu as pltpu
from jax.experimental.pallas import tpu_sc as plsc

EMBED = 64
HALF = EMBED // 2

_NC = 2
_NS = 16
_NW = _NC * _NS
_CHUNK = 1000


def _table_body(ts_ref, out_ref):
    t = ts_ref[:, :]
    col = lax.broadcasted_iota(jnp.int32, out_ref.shape, 1)
    is_sin = col < HALF
    k = jnp.where(is_sin, col, col - HALF).astype(jnp.float32)
    inv_freq = jnp.exp(k * (-2.0 * jnp.log(10000.0) / EMBED))
    phase = t * inv_freq
    out_ref[:, :] = jnp.where(is_sin, jnp.sin(phase), jnp.cos(phase))


def _build_table(time_step):
    g = time_step.shape[0]
    return pl.pallas_call(
        _table_body,
        out_shape=jax.ShapeDtypeStruct((g, EMBED), jnp.float32),
    )(time_step.reshape(g, 1))


def _gather_body(n_chunks, edge_hbm, batch_hbm, table_hbm, out_hbm,
                 idx_v, g_v, rows_v, sem_a, sem_b):
    wid = lax.axis_index("s") * _NC + lax.axis_index("c")
    base0 = wid * (n_chunks * _CHUNK)

    def step(i, carry):
        base = base0 + i * _CHUNK
        pltpu.sync_copy(edge_hbm.at[pl.ds(base, _CHUNK)], idx_v)
        pltpu.async_copy(batch_hbm.at[idx_v], g_v, sem_a).wait()
        pltpu.async_copy(table_hbm.at[g_v], rows_v, sem_b).wait()
        pltpu.sync_copy(rows_v, out_hbm.at[pl.ds(base, _CHUNK)])
        return carry

    lax.fori_loop(0, n_chunks, step, 0)


def _sc_gather(edge_row, batch, table):
    e = edge_row.shape[0]
    n_chunks = e // (_NW * _CHUNK)
    mesh = plsc.VectorSubcoreMesh(core_axis_name="c", subcore_axis_name="s")
    run = pl.kernel(
        functools.partial(_gather_body, n_chunks),
        out_type=jax.ShapeDtypeStruct((e, EMBED), jnp.float32),
        mesh=mesh,
        scratch_types=[
            pltpu.VMEM((_CHUNK,), jnp.int32),
            pltpu.VMEM((_CHUNK,), jnp.int32),
            pltpu.VMEM((_CHUNK, EMBED), jnp.float32),
            pltpu.SemaphoreType.DMA,
            pltpu.SemaphoreType.DMA,
        ],
        compiler_params=pltpu.CompilerParams(use_tc_tiling_on_sc=False),
    )
    return run(edge_row, batch, table)


def kernel(time_step, batch, edge_index):
    table = _build_table(time_step)
    edge_row = edge_index[0]
    e = edge_row.shape[0]
    tile = _NW * _CHUNK
    pad = (-e) % tile
    if pad:
        edge_row = jnp.concatenate(
            [edge_row, jnp.zeros((pad,), dtype=edge_row.dtype)])
    out = _sc_gather(edge_row, batch, table)
    if pad:
        out = out[:e]
    return out
